# Initial kernel scaffold; baseline (speedup 1.0000x reference)
import jax
import jax.numpy as jnp
from jax.experimental import pallas as pl


def kernel(x, edge_index, batch, W_gcn, b_gcn, w_in, b_in, w_out, b_out, W1, b1, W2, b2):
    raise NotImplementedError("write your pallas kernel here")


# speedup vs baseline: 12.2095x; 12.2095x over previous
import functools

import jax
import jax.numpy as jnp
from jax import lax
from jax.experimental import pallas as pl
from jax.experimental.pallas import tpu as pltpu
from jax.experimental.pallas import tpu_sc as plsc

_N = 10000
_D = 128
_EMB = 768
_NG = 64
_NOUT = 4
_NC, _NS = 2, 16
_NW = _NC * _NS
_CH = 128
_CPT = 80
_NCH = _NW * _CPT
_EP = _NCH * _CH
_NP = 10240
_STRIPE = _NP // _NS
_PAD_ROW = _N
_RB = 1024
_NBLK = _NP // _RB

_mesh = plsc.VectorSubcoreMesh(core_axis_name="c", subcore_axis_name="s",
                               num_cores=_NC, num_subcores=_NS)


_HALF = _NP // _NC
_NPL = 6144
_LSTRIPE = _NPL // _NS
_CPT2 = _NCH // _NS


def _remap_dst(dst_v, c):
    lo = c * _HALF

    def remap(j, carry):
        for g in range(_CH // 16):
            v = dst_v[j, pl.ds(g * 16, 16)] - lo
            ok = (v >= 0) & (v < _HALF)
            dst_v[j, pl.ds(g * 16, 16)] = jnp.where(ok, v, _HALF)
        return carry

    lax.fori_loop(0, _CPT2, remap, 0)


@functools.partial(
    pl.kernel,
    out_type=jax.ShapeDtypeStruct((_NC, _NPL, _D), jnp.float32),
    mesh=_mesh,
    scratch_types=[
        pltpu.VMEM((_CPT2, _CH), jnp.int32),
        pltpu.VMEM((_CH, _D), jnp.float32),
        pltpu.VMEM((_CH, _D), jnp.float32),
        pltpu.VMEM_SHARED((_NPL, _D), jnp.float32),
    ],
)
def _sc_degree(dst_hbm, ones_hbm, zeros_hbm, deg_hbm, dst_v, ones_v, zbuf_v,
               deg_sh):
    c = lax.axis_index("c")
    s = lax.axis_index("s")
    pltpu.sync_copy(dst_hbm.at[pl.ds(s * _CPT2, _CPT2)], dst_v)
    pltpu.sync_copy(ones_hbm, ones_v)
    pltpu.sync_copy(zeros_hbm, zbuf_v)
    base = s * _LSTRIPE
    for k in range(_LSTRIPE // _CH):
        pltpu.sync_copy(zbuf_v, deg_sh.at[pl.ds(base + k * _CH, _CH)])
    _remap_dst(dst_v, c)
    plsc.subcore_barrier()

    def body(j, carry):
        pltpu.sync_copy(ones_v, deg_sh.at[dst_v.at[j]], add=True)
        return carry

    lax.fori_loop(0, _CPT2, body, 0)
    plsc.subcore_barrier()
    for k in range(_LSTRIPE // _CH):
        off = base + k * _CH
        pltpu.sync_copy(deg_sh.at[pl.ds(off, _CH)], zbuf_v)
        pltpu.sync_copy(zbuf_v, deg_hbm.at[c, pl.ds(off, _CH)])


@functools.partial(
    pl.kernel,
    out_type=jax.ShapeDtypeStruct((_NC, _NPL, _D), jnp.float32),
    mesh=_mesh,
    scratch_types=[
        pltpu.VMEM((_CPT2, _CH), jnp.int32),
        pltpu.VMEM((_CPT2, _CH), jnp.int32),
        pltpu.VMEM((_CH, _D), jnp.float32),
        pltpu.VMEM((_CH, _D), jnp.float32),
        pltpu.VMEM_SHARED((_NPL, _D), jnp.float32),
        pltpu.SemaphoreType.DMA,
    ],
)
def _sc_scatter(src_hbm, dst_hbm, y_hbm, zeros_hbm, z_out_hbm, src_v, dst_v,
                rows_v, zbuf_v, z_sh, sem):
    c = lax.axis_index("c")
    s = lax.axis_index("s")
    pltpu.sync_copy(src_hbm.at[pl.ds(s * _CPT2, _CPT2)], src_v)
    pltpu.sync_copy(dst_hbm.at[pl.ds(s * _CPT2, _CPT2)], dst_v)
    pltpu.sync_copy(zeros_hbm, zbuf_v)
    base = s * _LSTRIPE
    for k in range(_LSTRIPE // _CH):
        pltpu.sync_copy(zbuf_v, z_sh.at[pl.ds(base + k * _CH, _CH)])
    _remap_dst(dst_v, c)
    plsc.subcore_barrier()

    def body(j, carry):
        pltpu.async_copy(y_hbm.at[src_v.at[j]], rows_v, sem).wait()
        pltpu.sync_copy(rows_v, z_sh.at[dst_v.at[j]], add=True)
        return carry

    lax.fori_loop(0, _CPT2, body, 0)
    plsc.subcore_barrier()
    for k in range(_LSTRIPE // _CH):
        off = base + k * _CH
        pltpu.sync_copy(z_sh.at[pl.ds(off, _CH)], rows_v)
        pltpu.sync_copy(rows_v, z_out_hbm.at[c, pl.ds(off, _CH)])


def _tc_scale_body(x_ref, deg_ref, y_ref, dinv_ref):
    d = deg_ref[0, :, 0:1] + 1.0
    dinv = lax.rsqrt(d)
    y_ref[...] = x_ref[...] * dinv
    dinv_ref[...] = jnp.broadcast_to(dinv, dinv_ref.shape)


_tc_scale = pl.pallas_call(
    _tc_scale_body,
    grid=(_NBLK,),
    in_specs=[
        pl.BlockSpec((_RB, _D), lambda i: (i, 0)),
        pl.BlockSpec((1, _RB, _D),
                     lambda i: (i // (_HALF // _RB), i % (_HALF // _RB), 0)),
    ],
    out_specs=[
        pl.BlockSpec((_RB, _D), lambda i: (i, 0)),
        pl.BlockSpec((_RB, 8), lambda i: (i, 0)),
    ],
    out_shape=[
        jax.ShapeDtypeStruct((_NP, _D), jnp.float32),
        jax.ShapeDtypeStruct((_NP, 8), jnp.float32),
    ],
)


def _tc_dense_body(zp_ref, y_ref, dinv_ref, batch_ref, wg_ref, bg_ref, wv_ref,
                   bv_ref, wo_ref, bo_ref, w1_ref, b1_ref, w2_ref, b2_ref,
                   out_ref, acc_ref):
    i = pl.program_id(0)

    @pl.when(i == 0)
    def _():
        acc_ref[...] = jnp.zeros_like(acc_ref)

    agg = (zp_ref[0] + y_ref[...]) * dinv_ref[:, 0:1]
    h = jnp.dot(agg, wg_ref[...], preferred_element_type=jnp.float32)
    h = jnp.maximum(h + bg_ref[...], 0.0)
    ids = lax.broadcasted_iota(jnp.int32, (_RB, _NG), 1)
    p = (batch_ref[...] == ids).astype(jnp.float32)
    haug = jnp.concatenate([h, jnp.ones((_RB, _D), jnp.float32)], axis=1)
    acc_ref[...] += lax.dot_general(
        p, haug, (((0,), (0,)), ((), ())), preferred_element_type=jnp.float32)

    @pl.when(i == _NBLK - 1)
    def _():
        acc = acc_ref[...]
        cnt = jnp.maximum(acc[:, _EMB:_EMB + 1], 1.0)
        g = acc[:, :_EMB] / cnt
        v = lax.dot_general(g, wv_ref[...], (((1,), (1,)), ((), ())),
                            preferred_element_type=jnp.float32) + bv_ref[...]
        a = lax.dot_general(v, wo_ref[...], (((1,), (1,)), ((), ())),
                            preferred_element_type=jnp.float32) + bo_ref[...]
        t = jnp.dot(a, w1_ref[...], preferred_element_type=jnp.float32)
        t = jnp.maximum(t + b1_ref[...], 0.0)
        o = jnp.dot(t, w2_ref[...],
                    preferred_element_type=jnp.float32) + b2_ref[...]
        m = jnp.max(o, axis=1, keepdims=True)
        e = jnp.exp(o - m)
        out_ref[...] = (o - m) - jnp.log(jnp.sum(e, axis=1, keepdims=True))


_tc_dense = pl.pallas_call(
    _tc_dense_body,
    grid=(_NBLK,),
    in_specs=[
        pl.BlockSpec((1, _RB, _D), lambda i: (i // (_HALF // _RB), i % (_HALF // _RB), 0)),
        pl.BlockSpec((_RB, _D), lambda i: (i, 0)),
        pl.BlockSpec((_RB, 8), lambda i: (i, 0)),
        pl.BlockSpec((_RB, 1), lambda i: (i, 0)),
        pl.BlockSpec((_D, _EMB), lambda i: (0, 0)),
        pl.BlockSpec((1, _EMB), lambda i: (0, 0)),
        pl.BlockSpec((_EMB, _EMB), lambda i: (0, 0)),
        pl.BlockSpec((1, _EMB), lambda i: (0, 0)),
        pl.BlockSpec((_EMB, _EMB), lambda i: (0, 0)),
        pl.BlockSpec((1, _EMB), lambda i: (0, 0)),
        pl.BlockSpec((_EMB, _EMB), lambda i: (0, 0)),
        pl.BlockSpec((1, _EMB), lambda i: (0, 0)),
        pl.BlockSpec((_EMB, _NOUT), lambda i: (0, 0)),
        pl.BlockSpec((1, _NOUT), lambda i: (0, 0)),
    ],
    out_specs=pl.BlockSpec((_NG, _NOUT), lambda i: (0, 0)),
    out_shape=jax.ShapeDtypeStruct((_NG, _NOUT), jnp.float32),
    scratch_shapes=[pltpu.VMEM((_NG, _EMB + _D), jnp.float32)],
)


def kernel(x, edge_index, batch, W_gcn, b_gcn, w_in, b_in, w_out, b_out, W1,
           b1, W2, b2):
    src = edge_index[0]
    dst = edge_index[1]
    padlen = _EP - src.shape[0]
    pad = jnp.full((padlen,), _PAD_ROW, jnp.int32)
    src_c = jnp.concatenate([src, pad]).reshape(_NCH, _CH)
    dst_c = jnp.concatenate([dst, pad]).reshape(_NCH, _CH)
    x_pad = jnp.pad(x, ((0, _NP - _N), (0, 0)))
    batch_pad = jnp.pad(batch, (0, _NP - _N),
                        constant_values=_NG).reshape(_NP, 1)
    ones128 = jnp.ones((_CH, _D), jnp.float32)
    zeros128 = jnp.zeros((_CH, _D), jnp.float32)

    degp = _sc_degree(dst_c, ones128, zeros128)
    y, dinv = _tc_scale(x_pad, degp)
    zp = _sc_scatter(src_c, dst_c, y, zeros128)
    wv = w_in[2 * _EMB:3 * _EMB]
    bv = b_in[2 * _EMB:3 * _EMB].reshape(1, _EMB)
    return _tc_dense(zp, y, dinv, batch_pad, W_gcn, b_gcn.reshape(1, _EMB),
                     wv, bv, w_out, b_out.reshape(1, _EMB), W1,
                     b1.reshape(1, _EMB), W2, b2.reshape(1, _NOUT))
